# baseline (speedup 1.0000x reference)
import jax
import jax.numpy as jnp
from jax.experimental import pallas as pl
from jax.experimental.pallas import tpu as pltpu

_BM = 200
_NSTEPS = 25


def _gcn_body(input_ref, weight_ref, adj_a_ref, adj_b_ref, out_ref,
              support_ref):
    @pl.when(pl.program_id(0) == 0)
    def _compute_support():
        x = input_ref[...].astype(jnp.bfloat16)
        w = weight_ref[...].astype(jnp.bfloat16)
        s = jnp.dot(x, w, preferred_element_type=jnp.float32)
        support_ref[...] = s.astype(jnp.bfloat16)

    s = support_ref[...]
    a = adj_a_ref[...].astype(jnp.bfloat16)
    b = adj_b_ref[...].astype(jnp.bfloat16)
    out_ref[0] = jnp.maximum(jnp.dot(a, s, preferred_element_type=jnp.float32), 0.0)
    out_ref[1] = jnp.maximum(jnp.dot(b, s, preferred_element_type=jnp.float32), 0.0)


def kernel(input, adj, weight):
    n, d_in = input.shape
    d_out = weight.shape[1]
    half = n // 2
    out = pl.pallas_call(
        _gcn_body,
        grid=(_NSTEPS,),
        in_specs=[
            pl.BlockSpec((n, d_in), lambda i: (0, 0)),
            pl.BlockSpec((d_in, d_out), lambda i: (0, 0)),
            pl.BlockSpec((_BM, n), lambda i: (i, 0)),
            pl.BlockSpec((_BM, n), lambda i: (i + _NSTEPS, 0)),
        ],
        out_specs=pl.BlockSpec((2, _BM, d_out), lambda i: (0, i, 0)),
        out_shape=jax.ShapeDtypeStruct((2, half, d_out), jnp.float32),
        scratch_shapes=[pltpu.VMEM((n, d_out), jnp.bfloat16)],
    )(input.astype(jnp.float32), weight, adj, adj)
    return out.reshape(n, d_out)
